# baseline (speedup 1.0000x reference)
import functools

import jax
import jax.numpy as jnp
from jax import lax
from jax.experimental import pallas as pl
from jax.experimental.pallas import tpu as pltpu
from jax.experimental.pallas import tpu_sc as plsc

N = 10000
E = 320000
DF = 128
DE = 16
D = 64
H = 128
G = 64

LCH = 128
NW = 32
BPW = 10240
CPW = BPW // LCH
E_PAD = NW * BPW
N_ACC = 10240
NZB = N_ACC // 16
EB = 2048


def _mesh():
    return plsc.VectorSubcoreMesh(
        core_axis_name="c", subcore_axis_name="s", num_cores=2, num_subcores=16
    )


_SC_PARAMS = pltpu.CompilerParams(use_tc_tiling_on_sc=False)


def _sp(v):
    return jnp.maximum(v, 0.0) + jnp.log(1.0 + jnp.exp(-jnp.abs(v)))


def _mm(a, b):
    return lax.dot_general(a, b, (((1,), (1,)), ((), ())),
                           preferred_element_type=jnp.float32,
                           precision=lax.Precision.HIGHEST)


def _sc_gather2(xt, ir2, ic2):
    kern = functools.partial(
        pl.kernel,
        out_type=(jax.ShapeDtypeStruct((E_PAD, D), jnp.float32),
                  jax.ShapeDtypeStruct((E_PAD, D), jnp.float32)),
        mesh=_mesh(),
        scratch_types=[pltpu.VMEM((CPW, LCH), jnp.int32),
                       pltpu.VMEM((CPW, LCH), jnp.int32),
                       pltpu.VMEM((LCH, D), jnp.float32),
                       pltpu.VMEM((LCH, D), jnp.float32),
                       pltpu.SemaphoreType.DMA,
                       pltpu.SemaphoreType.DMA],
        compiler_params=_SC_PARAMS,
    )

    @kern
    def k(x_hbm, ir_hbm, ic_hbm, or_hbm, oc_hbm, ir_v, ic_v, br_v, bc_v, s1, s2):
        wid = lax.axis_index("c") * 16 + lax.axis_index("s")
        pltpu.sync_copy(ir_hbm.at[pl.ds(wid * CPW, CPW)], ir_v)
        pltpu.sync_copy(ic_hbm.at[pl.ds(wid * CPW, CPW)], ic_v)

        @pl.loop(0, CPW)
        def _(j):
            c1 = pltpu.async_copy(x_hbm.at[ir_v.at[j]], br_v, s1)
            c2 = pltpu.async_copy(x_hbm.at[ic_v.at[j]], bc_v, s2)
            c1.wait()
            c2.wait()
            base = wid * BPW + j * LCH
            pltpu.sync_copy(br_v, or_hbm.at[pl.ds(base, LCH)])
            pltpu.sync_copy(bc_v, oc_hbm.at[pl.ds(base, LCH)])

    return k(xt, ir2, ic2)


def _sc_scatter(m, ic2, zrows):
    kern = functools.partial(
        pl.kernel,
        out_type=jax.ShapeDtypeStruct((2 * N_ACC, D), jnp.float32),
        mesh=_mesh(),
        scratch_types=[pltpu.VMEM((CPW, LCH), jnp.int32),
                       pltpu.VMEM((LCH, D), jnp.float32),
                       pltpu.VMEM_SHARED((N_ACC, D), jnp.float32),
                       pltpu.SemaphoreType.DMA],
        compiler_params=_SC_PARAMS,
    )

    @kern
    def k(m_hbm, ic_hbm, z_hbm, out_hbm, iv, bv, acc, sem):
        cid = lax.axis_index("c")
        sid = lax.axis_index("s")
        pltpu.sync_copy(z_hbm, acc.at[pl.ds(sid * NZB, NZB)])
        plsc.subcore_barrier()
        wid = cid * 16 + sid
        pltpu.sync_copy(ic_hbm.at[pl.ds(wid * CPW, CPW)], iv)

        @pl.loop(0, CPW)
        def _(j):
            pltpu.sync_copy(m_hbm.at[pl.ds(wid * BPW + j * LCH, LCH)], bv)
            pltpu.sync_copy(bv, acc.at[iv.at[j]], add=True)

        plsc.subcore_barrier()
        pltpu.sync_copy(acc.at[pl.ds(sid * NZB, NZB)],
                        out_hbm.at[pl.ds(cid * N_ACC + sid * NZB, NZB)])

    return k(m, ic2, zrows)


def _embed_x(x, atom_W, atom_b):
    def body(x_ref, w_ref, b_ref, o_ref):
        o_ref[...] = _mm(x_ref[...], w_ref[...]) + b_ref[...]

    return pl.pallas_call(
        body, out_shape=jax.ShapeDtypeStruct((N, D), jnp.float32)
    )(x, atom_W, atom_b)


def _embed_ea(ea_p, bond_W, bond_b):
    def body(a_ref, w_ref, b_ref, o_ref):
        o_ref[...] = _mm(a_ref[...], w_ref[...]) + b_ref[...]

    return pl.pallas_call(
        body, grid=(E_PAD // EB,),
        in_specs=[pl.BlockSpec((EB, DE), lambda i: (i, 0)),
                  pl.BlockSpec((D, DE), lambda i: (0, 0)),
                  pl.BlockSpec((1, D), lambda i: (0, 0))],
        out_specs=pl.BlockSpec((EB, D), lambda i: (i, 0)),
        out_shape=jax.ShapeDtypeStruct((E_PAD, D), jnp.float32),
    )(ea_p, bond_W, bond_b)


def _edge_mlp(xr, xc, ea, euW1, eub1, euW2, eub2, nuW1, nub1, nuW2, nub2):
    def body(xr_ref, xc_ref, ea_ref, w1, b1, w2, b2, nw1, nb1, nw2, nb2,
             ea_o, m_o):
        xr_ = xr_ref[...]
        xc_ = xc_ref[...]
        ea_ = ea_ref[...]
        w1_ = w1[...]
        t = (_mm(xr_, w1_[:, :D]) + _mm(xc_, w1_[:, D:2 * D])
             + _mm(ea_, w1_[:, 2 * D:]) + b1[...])
        ea2 = _mm(_sp(t), w2[...]) + b2[...]
        ea_o[...] = ea2
        nw1_ = nw1[...]
        u = _mm(xr_, nw1_[:, :D]) + _mm(ea2, nw1_[:, D:]) + nb1[...]
        m_o[...] = _mm(_sp(u), nw2[...]) + nb2[...]

    eb_spec = pl.BlockSpec((EB, D), lambda i: (i, 0))

    def w_spec(shp):
        return pl.BlockSpec(shp, lambda i: (0, 0))

    return pl.pallas_call(
        body, grid=(E_PAD // EB,),
        in_specs=[eb_spec, eb_spec, eb_spec,
                  w_spec((D, 3 * D)), w_spec((1, D)), w_spec((D, D)),
                  w_spec((1, D)), w_spec((D, 2 * D)), w_spec((1, D)),
                  w_spec((D, D)), w_spec((1, D))],
        out_specs=[eb_spec, eb_spec],
        out_shape=[jax.ShapeDtypeStruct((E_PAD, D), jnp.float32)] * 2,
    )(xr, xc, ea, euW1, eub1, euW2, eub2, nuW1, nub1, nuW2, nub2)


def _update_x(part, x_old, bng, bnb):
    def body(p_ref, x_ref, g_ref, b_ref, o_ref):
        agg = p_ref[0:N, :] + p_ref[N_ACC:N_ACC + N, :]
        scale = 1.0 / jnp.sqrt(jnp.float32(1.0 + 1e-5))
        xn = agg * scale * g_ref[...] + b_ref[...]
        o_ref[...] = _sp(xn) + x_ref[...]

    return pl.pallas_call(
        body, out_shape=jax.ShapeDtypeStruct((N, D), jnp.float32)
    )(part, x_old, bng, bnb)


def _pool(xf, batch_row, pW1, pb1, pW2, pb2, pW3, pb3):
    def body(x_ref, b_ref, w1, b1, w2, b2, w3, b3, o_ref):
        ids = b_ref[...]
        gi = lax.broadcasted_iota(jnp.int32, (G, 1), 0)
        oh = (gi == ids).astype(jnp.float32)
        cnt = jnp.sum(oh, axis=1, keepdims=True)
        ohs = oh / jnp.maximum(cnt, 1.0)
        g = lax.dot_general(ohs, x_ref[...], (((1,), (0,)), ((), ())),
                            preferred_element_type=jnp.float32,
                            precision=lax.Precision.HIGHEST)
        h = _sp(_mm(g, w1[...]) + b1[...])
        h = _sp(_mm(h, w2[...]) + b2[...])
        o_ref[...] = _mm(h, w3[...]) + b3[0, 0]

    return pl.pallas_call(
        body, out_shape=jax.ShapeDtypeStruct((G, 8), jnp.float32)
    )(xf, batch_row, pW1, pb1, pW2, pb2, pW3, pb3)


def kernel(x, edge_attr, edge_index, batch, atom_W, atom_b, bond_W, bond_b,
           c0_nuW1, c0_nub1, c0_nuW2, c0_nub2, c0_euW1, c0_eub1, c0_euW2,
           c0_eub2, c0_bng, c0_bnb, c1_nuW1, c1_nub1, c1_nuW2, c1_nub2,
           c1_euW1, c1_eub1, c1_euW2, c1_eub2, c1_bng, c1_bnb, c2_nuW1,
           c2_nub1, c2_nuW2, c2_nub2, c2_euW1, c2_eub1, c2_euW2, c2_eub2,
           c2_bng, c2_bnb, pW1, pb1, pW2, pb2, pW3, pb3):
    f32 = jnp.float32
    pad = E_PAD - E
    row = edge_index[0]
    col = edge_index[1]
    ir2 = jnp.concatenate([row, jnp.zeros((pad,), jnp.int32)]).reshape(
        E_PAD // LCH, LCH)
    ic2 = jnp.concatenate([col, jnp.full((pad,), N, jnp.int32)]).reshape(
        E_PAD // LCH, LCH)
    ea_p = jnp.concatenate([edge_attr, jnp.zeros((pad, DE), f32)], axis=0)
    zrows = jnp.zeros((NZB, D), f32)
    batch_row = batch.reshape(1, N)

    xb = _embed_x(x, atom_W, atom_b.reshape(1, D))
    ea = _embed_ea(ea_p, bond_W, bond_b.reshape(1, D))

    layers = [
        (c0_nuW1, c0_nub1, c0_nuW2, c0_nub2, c0_euW1, c0_eub1, c0_euW2,
         c0_eub2, c0_bng, c0_bnb),
        (c1_nuW1, c1_nub1, c1_nuW2, c1_nub2, c1_euW1, c1_eub1, c1_euW2,
         c1_eub2, c1_bng, c1_bnb),
        (c2_nuW1, c2_nub1, c2_nuW2, c2_nub2, c2_euW1, c2_eub1, c2_euW2,
         c2_eub2, c2_bng, c2_bnb),
    ]
    for (nuW1, nub1, nuW2, nub2, euW1, eub1, euW2, eub2, bng, bnb) in layers:
        xr, xc = _sc_gather2(xb, ir2, ic2)
        ea, m = _edge_mlp(xr, xc, ea, euW1, eub1.reshape(1, D), euW2,
                          eub2.reshape(1, D), nuW1, nub1.reshape(1, D),
                          nuW2, nub2.reshape(1, D))
        part = _sc_scatter(m, ic2, zrows)
        xb = _update_x(part, xb, bng.reshape(1, D), bnb.reshape(1, D))

    pW3p = jnp.zeros((8, H), f32).at[0].set(pW3[0])
    out = _pool(xb, batch_row, pW1, pb1.reshape(1, H), pW2, pb2.reshape(1, H),
                pW3p, pb3.reshape(1, 1))
    return out[:, 0]


# speedup vs baseline: 1.3909x; 1.3909x over previous
import functools

import jax
import jax.numpy as jnp
from jax import lax
from jax.experimental import pallas as pl
from jax.experimental.pallas import tpu as pltpu
from jax.experimental.pallas import tpu_sc as plsc

N = 10000
E = 320000
DF = 128
DE = 16
D = 64
H = 128
G = 64

LCH = 128
NW = 32
BPW = 10240
CPW = BPW // LCH
E_PAD = NW * BPW
N_ACC = 10240
NZB = N_ACC // 16
EB = 2048
MCH = 256
CPM = MCH // LCH
NM = BPW // MCH


def _mesh():
    return plsc.VectorSubcoreMesh(
        core_axis_name="c", subcore_axis_name="s", num_cores=2, num_subcores=16
    )


_SC_PARAMS = pltpu.CompilerParams(use_tc_tiling_on_sc=False)


def _sp(v):
    return jnp.maximum(v, 0.0) + jnp.log(1.0 + jnp.exp(-jnp.abs(v)))


def _mm(a, b):
    return lax.dot_general(a, b, (((1,), (1,)), ((), ())),
                           preferred_element_type=jnp.float32,
                           precision=lax.Precision.HIGHEST)


def _sc_gather2(xt, ir2, ic2):
    kern = functools.partial(
        pl.kernel,
        out_type=(jax.ShapeDtypeStruct((E_PAD, D), jnp.float32),
                  jax.ShapeDtypeStruct((E_PAD, D), jnp.float32)),
        mesh=_mesh(),
        scratch_types=[pltpu.VMEM((CPW, LCH), jnp.int32),
                       pltpu.VMEM((CPW, LCH), jnp.int32),
                       pltpu.VMEM((2, MCH, D), jnp.float32),
                       pltpu.VMEM((2, MCH, D), jnp.float32),
                       pltpu.SemaphoreType.DMA((2,)),
                       pltpu.SemaphoreType.DMA((2,)),
                       pltpu.SemaphoreType.DMA((2,)),
                       pltpu.SemaphoreType.DMA((2,))],
        compiler_params=_SC_PARAMS,
    )

    @kern
    def k(x_hbm, ir_hbm, ic_hbm, or_hbm, oc_hbm, ir_v, ic_v, br_v, bc_v,
          gr, gc, wr, wc):
        wid = lax.axis_index("c") * 16 + lax.axis_index("s")
        pltpu.sync_copy(ir_hbm.at[pl.ds(wid * CPW, CPW)], ir_v)
        pltpu.sync_copy(ic_hbm.at[pl.ds(wid * CPW, CPW)], ic_v)

        def fire(m, s):
            for q in range(CPM):
                j = m * CPM + q
                pltpu.async_copy(x_hbm.at[ir_v.at[j]],
                                 br_v.at[s, pl.ds(q * LCH, LCH)], gr.at[s])
                pltpu.async_copy(x_hbm.at[ic_v.at[j]],
                                 bc_v.at[s, pl.ds(q * LCH, LCH)], gc.at[s])

        def wait_g(m, s):
            for q in range(CPM):
                j = m * CPM + q
                pltpu.make_async_copy(x_hbm.at[ir_v.at[j]],
                                      br_v.at[s, pl.ds(q * LCH, LCH)],
                                      gr.at[s]).wait()
                pltpu.make_async_copy(x_hbm.at[ic_v.at[j]],
                                      bc_v.at[s, pl.ds(q * LCH, LCH)],
                                      gc.at[s]).wait()

        def out_slc(m):
            return pl.ds(wid * BPW + m * MCH, MCH)

        def fire_w(m, s):
            pltpu.async_copy(br_v.at[s], or_hbm.at[out_slc(m)], wr.at[s])
            pltpu.async_copy(bc_v.at[s], oc_hbm.at[out_slc(m)], wc.at[s])

        def wait_w(m, s):
            pltpu.make_async_copy(br_v.at[s], or_hbm.at[out_slc(m)],
                                  wr.at[s]).wait()
            pltpu.make_async_copy(bc_v.at[s], oc_hbm.at[out_slc(m)],
                                  wc.at[s]).wait()

        fire(0, 0)
        fire(1, 1)

        @pl.loop(0, NM - 2, step=2)
        def _(m):
            wait_g(m, 0)
            fire_w(m, 0)
            wait_w(m, 0)
            fire(m + 2, 0)
            wait_g(m + 1, 1)
            fire_w(m + 1, 1)
            wait_w(m + 1, 1)
            fire(m + 3, 1)

        wait_g(NM - 2, 0)
        fire_w(NM - 2, 0)
        wait_g(NM - 1, 1)
        fire_w(NM - 1, 1)
        wait_w(NM - 2, 0)
        wait_w(NM - 1, 1)

    return k(xt, ir2, ic2)


def _sc_scatter(m, ic2, zrows):
    kern = functools.partial(
        pl.kernel,
        out_type=jax.ShapeDtypeStruct((2 * N_ACC, D), jnp.float32),
        mesh=_mesh(),
        scratch_types=[pltpu.VMEM((CPW, LCH), jnp.int32),
                       pltpu.VMEM((2, LCH, D), jnp.float32),
                       pltpu.VMEM_SHARED((N_ACC, D), jnp.float32),
                       pltpu.SemaphoreType.DMA((2,))],
        compiler_params=_SC_PARAMS,
    )

    @kern
    def k(m_hbm, ic_hbm, z_hbm, out_hbm, iv, bv, acc, lm):
        cid = lax.axis_index("c")
        sid = lax.axis_index("s")
        pltpu.sync_copy(z_hbm, acc.at[pl.ds(sid * NZB, NZB)])
        plsc.subcore_barrier()
        wid = cid * 16 + sid
        pltpu.sync_copy(ic_hbm.at[pl.ds(wid * CPW, CPW)], iv)

        def m_slc(j):
            return pl.ds(wid * BPW + j * LCH, LCH)

        def fire_ld(j, s):
            pltpu.async_copy(m_hbm.at[m_slc(j)], bv.at[s], lm.at[s])

        def wait_ld(j, s):
            pltpu.make_async_copy(m_hbm.at[m_slc(j)], bv.at[s],
                                  lm.at[s]).wait()

        fire_ld(0, 0)
        fire_ld(1, 1)

        @pl.loop(0, CPW - 2, step=2)
        def _(j):
            wait_ld(j, 0)
            pltpu.sync_copy(bv.at[0], acc.at[iv.at[j]], add=True)
            fire_ld(j + 2, 0)
            wait_ld(j + 1, 1)
            pltpu.sync_copy(bv.at[1], acc.at[iv.at[j + 1]], add=True)
            fire_ld(j + 3, 1)

        wait_ld(CPW - 2, 0)
        pltpu.sync_copy(bv.at[0], acc.at[iv.at[CPW - 2]], add=True)
        wait_ld(CPW - 1, 1)
        pltpu.sync_copy(bv.at[1], acc.at[iv.at[CPW - 1]], add=True)

        plsc.subcore_barrier()
        pltpu.sync_copy(acc.at[pl.ds(sid * NZB, NZB)],
                        out_hbm.at[pl.ds(cid * N_ACC + sid * NZB, NZB)])

    return k(m, ic2, zrows)


def _embed_x(x, atom_W, atom_b):
    def body(x_ref, w_ref, b_ref, o_ref):
        o_ref[...] = _mm(x_ref[...], w_ref[...]) + b_ref[...]

    return pl.pallas_call(
        body, out_shape=jax.ShapeDtypeStruct((N, D), jnp.float32)
    )(x, atom_W, atom_b)


def _embed_ea(ea_p, bond_W, bond_b):
    def body(a_ref, w_ref, b_ref, o_ref):
        o_ref[...] = _mm(a_ref[...], w_ref[...]) + b_ref[...]

    return pl.pallas_call(
        body, grid=(E_PAD // EB,),
        in_specs=[pl.BlockSpec((EB, DE), lambda i: (i, 0)),
                  pl.BlockSpec((D, DE), lambda i: (0, 0)),
                  pl.BlockSpec((1, D), lambda i: (0, 0))],
        out_specs=pl.BlockSpec((EB, D), lambda i: (i, 0)),
        out_shape=jax.ShapeDtypeStruct((E_PAD, D), jnp.float32),
    )(ea_p, bond_W, bond_b)


def _prep_wbig(euW2, nuW1, eub2, nub1):
    def body(w2_ref, nw1_ref, b2_ref, nb1_ref, wb_ref, bb_ref):
        w2 = w2_ref[...]
        nw1 = nw1_ref[...]
        a = nw1[:, :D]
        b = nw1[:, D:]
        C = lax.dot_general(b, w2, (((1,), (0,)), ((), ())),
                            preferred_element_type=jnp.float32,
                            precision=lax.Precision.HIGHEST)
        top = jnp.concatenate([jnp.zeros((D, D), jnp.float32), w2], axis=1)
        bot = jnp.concatenate([a, C], axis=1)
        wb_ref[...] = jnp.concatenate([top, bot], axis=0)
        bb_ref[...] = jnp.concatenate(
            [b2_ref[...], nb1_ref[...] + _mm(b2_ref[...], b)], axis=1)

    return pl.pallas_call(
        body, out_shape=[jax.ShapeDtypeStruct((2 * D, 2 * D), jnp.float32),
                         jax.ShapeDtypeStruct((1, 2 * D), jnp.float32)]
    )(euW2, nuW1, eub2, nub1)


def _edge_mlp(xr, xc, ea, euW1, eub1, wbig, bbig, nuW2, nub2):
    def body(xr_ref, xc_ref, ea_ref, w1, b1, wb, bb, nw2, nb2, ea_o, m_o):
        xr_ = xr_ref[...]
        cat = jnp.concatenate([xr_, xc_ref[...], ea_ref[...]], axis=1)
        h = _sp(_mm(cat, w1[...]) + b1[...])
        cat2 = jnp.concatenate([xr_, h], axis=1)
        big = _mm(cat2, wb[...]) + bb[...]
        ea_o[...] = big[:, :D]
        m_o[...] = _mm(_sp(big[:, D:]), nw2[...]) + nb2[...]

    eb_spec = pl.BlockSpec((EB, D), lambda i: (i, 0))

    def w_spec(shp):
        return pl.BlockSpec(shp, lambda i: (0, 0))

    return pl.pallas_call(
        body, grid=(E_PAD // EB,),
        in_specs=[eb_spec, eb_spec, eb_spec,
                  w_spec((D, 3 * D)), w_spec((1, D)),
                  w_spec((2 * D, 2 * D)), w_spec((1, 2 * D)),
                  w_spec((D, D)), w_spec((1, D))],
        out_specs=[eb_spec, eb_spec],
        out_shape=[jax.ShapeDtypeStruct((E_PAD, D), jnp.float32)] * 2,
    )(xr, xc, ea, euW1, eub1, wbig, bbig, nuW2, nub2)


def _update_x(part, x_old, bng, bnb):
    def body(p_ref, x_ref, g_ref, b_ref, o_ref):
        agg = p_ref[0:N, :] + p_ref[N_ACC:N_ACC + N, :]
        scale = 1.0 / jnp.sqrt(jnp.float32(1.0 + 1e-5))
        xn = agg * scale * g_ref[...] + b_ref[...]
        o_ref[...] = _sp(xn) + x_ref[...]

    return pl.pallas_call(
        body, out_shape=jax.ShapeDtypeStruct((N, D), jnp.float32)
    )(part, x_old, bng, bnb)


def _pool(xf, batch_row, pW1, pb1, pW2, pb2, pW3, pb3):
    def body(x_ref, b_ref, w1, b1, w2, b2, w3, b3, o_ref):
        ids = b_ref[...]
        gi = lax.broadcasted_iota(jnp.int32, (G, 1), 0)
        oh = (gi == ids).astype(jnp.float32)
        cnt = jnp.sum(oh, axis=1, keepdims=True)
        ohs = oh / jnp.maximum(cnt, 1.0)
        g = lax.dot_general(ohs, x_ref[...], (((1,), (0,)), ((), ())),
                            preferred_element_type=jnp.float32,
                            precision=lax.Precision.HIGHEST)
        h = _sp(_mm(g, w1[...]) + b1[...])
        h = _sp(_mm(h, w2[...]) + b2[...])
        o_ref[...] = _mm(h, w3[...]) + b3[0, 0]

    return pl.pallas_call(
        body, out_shape=jax.ShapeDtypeStruct((G, 8), jnp.float32)
    )(xf, batch_row, pW1, pb1, pW2, pb2, pW3, pb3)


def kernel(x, edge_attr, edge_index, batch, atom_W, atom_b, bond_W, bond_b,
           c0_nuW1, c0_nub1, c0_nuW2, c0_nub2, c0_euW1, c0_eub1, c0_euW2,
           c0_eub2, c0_bng, c0_bnb, c1_nuW1, c1_nub1, c1_nuW2, c1_nub2,
           c1_euW1, c1_eub1, c1_euW2, c1_eub2, c1_bng, c1_bnb, c2_nuW1,
           c2_nub1, c2_nuW2, c2_nub2, c2_euW1, c2_eub1, c2_euW2, c2_eub2,
           c2_bng, c2_bnb, pW1, pb1, pW2, pb2, pW3, pb3):
    f32 = jnp.float32
    pad = E_PAD - E
    row = edge_index[0]
    col = edge_index[1]
    ir2 = jnp.concatenate([row, jnp.zeros((pad,), jnp.int32)]).reshape(
        E_PAD // LCH, LCH)
    ic2 = jnp.concatenate([col, jnp.full((pad,), N, jnp.int32)]).reshape(
        E_PAD // LCH, LCH)
    ea_p = jnp.concatenate([edge_attr, jnp.zeros((pad, DE), f32)], axis=0)
    zrows = jnp.zeros((NZB, D), f32)
    batch_row = batch.reshape(1, N)

    xb = _embed_x(x, atom_W, atom_b.reshape(1, D))
    ea = _embed_ea(ea_p, bond_W, bond_b.reshape(1, D))

    layers = [
        (c0_nuW1, c0_nub1, c0_nuW2, c0_nub2, c0_euW1, c0_eub1, c0_euW2,
         c0_eub2, c0_bng, c0_bnb),
        (c1_nuW1, c1_nub1, c1_nuW2, c1_nub2, c1_euW1, c1_eub1, c1_euW2,
         c1_eub2, c1_bng, c1_bnb),
        (c2_nuW1, c2_nub1, c2_nuW2, c2_nub2, c2_euW1, c2_eub1, c2_euW2,
         c2_eub2, c2_bng, c2_bnb),
    ]
    for (nuW1, nub1, nuW2, nub2, euW1, eub1, euW2, eub2, bng, bnb) in layers:
        wbig, bbig = _prep_wbig(euW2, nuW1, eub2.reshape(1, D),
                                nub1.reshape(1, D))
        xr, xc = _sc_gather2(xb, ir2, ic2)
        ea, m = _edge_mlp(xr, xc, ea, euW1, eub1.reshape(1, D), wbig, bbig,
                          nuW2, nub2.reshape(1, D))
        part = _sc_scatter(m, ic2, zrows)
        xb = _update_x(part, xb, bng.reshape(1, D), bnb.reshape(1, D))

    pW3p = jnp.zeros((8, H), f32).at[0].set(pW3[0])
    out = _pool(xb, batch_row, pW1, pb1.reshape(1, H), pW2, pb2.reshape(1, H),
                pW3p, pb3.reshape(1, 1))
    return out[:, 0]
